# baseline (speedup 1.0000x reference)
import functools

import jax
import jax.numpy as jnp
from jax import lax
from jax.experimental import pallas as pl
from jax.experimental.pallas import tpu as pltpu
from jax.experimental.pallas import tpu_sc as plsc

MAP_H, MAP_W = 32, 32
N = MAP_H * MAP_W
D = 128
B = 1024

_NC, _NS = 2, 16
_NW = _NC * _NS
_BPW = B // _NW


_BCHUNK = 256


def _argmin_body(z_ref, w_ref, idx_ref):
    z = z_ref[:]
    w = w_ref[:]
    dot = lax.dot_general(
        w, z, (((1,), (1,)), ((), ())),
        preferred_element_type=jnp.float32,
        precision=lax.Precision.HIGHEST,
    )
    wsq = jnp.sum(w * w, axis=1, keepdims=True)
    scores = wsq - 2.0 * dot
    minval = jnp.min(scores, axis=0, keepdims=True)
    rowid = lax.broadcasted_iota(jnp.int32, (N, _BCHUNK), 0)
    idx = jnp.min(jnp.where(scores == minval, rowid, jnp.int32(N)), axis=0)
    idx_ref[:] = idx


_argmin_call = pl.pallas_call(
    _argmin_body,
    grid=(B // _BCHUNK,),
    in_specs=[
        pl.BlockSpec((_BCHUNK, D), lambda i: (i, 0)),
        pl.BlockSpec((N, D), lambda i: (0, 0)),
    ],
    out_specs=pl.BlockSpec((_BCHUNK,), lambda i: (i,)),
    out_shape=jax.ShapeDtypeStruct((B,), jnp.int32),
)


@functools.cache
def _bmu_gather_call():
    mesh = plsc.VectorSubcoreMesh(
        core_axis_name="c", subcore_axis_name="s", num_cores=_NC)

    @functools.partial(
        pl.kernel,
        mesh=mesh,
        out_type=jax.ShapeDtypeStruct((B, D), jnp.float32),
        scratch_types=[
            pltpu.VMEM((_BPW,), jnp.int32),
            pltpu.VMEM((_BPW, D), jnp.float32),
            pltpu.SemaphoreType.DMA,
        ],
    )
    def _bmu_gather(table_hbm, idx_hbm, out_hbm, idx_v, rows_v, sem):
        wid = lax.axis_index("s") * _NC + lax.axis_index("c")
        base = wid * _BPW
        pltpu.sync_copy(idx_hbm.at[pl.ds(base, _BPW)], idx_v)
        pltpu.async_copy(table_hbm.at[idx_v], rows_v, sem).wait()
        pltpu.sync_copy(rows_v, out_hbm.at[pl.ds(base, _BPW)])

    return _bmu_gather


def kernel(inputs, SOM):
    table = SOM.reshape(N, D)
    idx = _argmin_call(inputs, table)
    return _bmu_gather_call()(table, idx)


# speedup vs baseline: 1.0848x; 1.0848x over previous
import functools

import jax
import jax.numpy as jnp
from jax import lax
from jax.experimental import pallas as pl
from jax.experimental.pallas import tpu as pltpu
from jax.experimental.pallas import tpu_sc as plsc

MAP_H, MAP_W = 32, 32
N = MAP_H * MAP_W
D = 128
B = 1024

_NC, _NS = 1, 16
_NW = _NC * _NS
_BPW = B // _NW


def _argmin_body(z_ref, w_ref, idx_ref):
    z = z_ref[:]
    w = w_ref[:]
    dot = lax.dot_general(
        w, z, (((1,), (1,)), ((), ())),
        preferred_element_type=jnp.float32,
        precision=lax.Precision.HIGHEST,
    )
    wsq = jnp.sum(w * w, axis=1, keepdims=True)
    scores = wsq - 2.0 * dot
    minval = jnp.min(scores, axis=0, keepdims=True)
    rowid = lax.broadcasted_iota(jnp.int32, (N, B), 0)
    idx = jnp.min(jnp.where(scores == minval, rowid, jnp.int32(N)), axis=0)
    idx_ref[:] = idx


_argmin_call = pl.pallas_call(
    _argmin_body,
    out_shape=jax.ShapeDtypeStruct((B,), jnp.int32),
)


@functools.cache
def _bmu_gather_call():
    mesh = plsc.VectorSubcoreMesh(
        core_axis_name="c", subcore_axis_name="s", num_cores=_NC)

    @functools.partial(
        pl.kernel,
        mesh=mesh,
        out_type=jax.ShapeDtypeStruct((B, D), jnp.float32),
        scratch_types=[
            pltpu.VMEM((_BPW,), jnp.int32),
            pltpu.VMEM((_BPW, D), jnp.float32),
            pltpu.SemaphoreType.DMA,
        ],
    )
    def _bmu_gather(table_hbm, idx_hbm, out_hbm, idx_v, rows_v, sem):
        wid = lax.axis_index("s") * _NC + lax.axis_index("c")
        base = wid * _BPW
        pltpu.sync_copy(idx_hbm.at[pl.ds(base, _BPW)], idx_v)
        pltpu.async_copy(table_hbm.at[idx_v], rows_v, sem).wait()
        pltpu.sync_copy(rows_v, out_hbm.at[pl.ds(base, _BPW)])

    return _bmu_gather


def kernel(inputs, SOM):
    table = SOM.reshape(N, D)
    idx = _argmin_call(inputs, table)
    return _bmu_gather_call()(table, idx)
